# baseline (speedup 1.0000x reference)
import functools

import jax
import jax.numpy as jnp
from jax import lax
from jax.experimental import pallas as pl
from jax.experimental.pallas import tpu as pltpu
from jax.experimental.pallas import tpu_sc as plsc

_VOCAB = 100000
_EMBED = 64
_BATCH = 1024
_BLOCK_V = 2048
_NBLK = (_VOCAB + _BLOCK_V - 1) // _BLOCK_V


def _sc_gather(target, in_emb):
    info = plsc.get_sparse_core_info()
    n_workers = info.num_cores * info.num_subcores
    bpw = _BATCH // n_workers
    mesh = plsc.VectorSubcoreMesh(core_axis_name="c", subcore_axis_name="s")

    @functools.partial(
        pl.kernel,
        mesh=mesh,
        out_type=jax.ShapeDtypeStruct((_BATCH, _EMBED), jnp.float32),
        scratch_types=[
            pltpu.VMEM((bpw,), jnp.int32),
            pltpu.VMEM((bpw, _EMBED), jnp.float32),
            pltpu.SemaphoreType.DMA,
        ],
        compiler_params=pltpu.CompilerParams(use_tc_tiling_on_sc=False),
    )
    def gather_kernel(table_hbm, idx_hbm, out_hbm, idx_v, rows_v, sem):
        wid = lax.axis_index("s") * info.num_cores + lax.axis_index("c")
        base = wid * bpw
        pltpu.sync_copy(idx_hbm.at[pl.ds(base, bpw)], idx_v)
        pltpu.async_copy(table_hbm.at[idx_v], rows_v, sem).wait()
        pltpu.sync_copy(rows_v, out_hbm.at[pl.ds(base, bpw)])

    return gather_kernel(in_emb, target)


def _tc_logits_t(embed, out_emb):
    def body(w_ref, emb_ref, out_ref):
        out_ref[...] = lax.dot_general(
            w_ref[...],
            emb_ref[...],
            dimension_numbers=(((1,), (1,)), ((), ())),
            preferred_element_type=jnp.float32,
        )

    return pl.pallas_call(
        body,
        grid=(_NBLK,),
        in_specs=[
            pl.BlockSpec((_BLOCK_V, _EMBED), lambda i: (i, 0)),
            pl.BlockSpec((_BATCH, _EMBED), lambda i: (0, 0)),
        ],
        out_specs=pl.BlockSpec((_BLOCK_V, _BATCH), lambda i: (i, 0)),
        out_shape=jax.ShapeDtypeStruct((_VOCAB, _BATCH), jnp.float32),
        compiler_params=pltpu.CompilerParams(
            dimension_semantics=("arbitrary",),
        ),
    )(out_emb, embed)


def kernel(target, in_emb, out_emb):
    embed = _sc_gather(target.astype(jnp.int32), in_emb)
    return _tc_logits_t(embed, out_emb).T


# speedup vs baseline: 1.0124x; 1.0124x over previous
import functools

import jax
import jax.numpy as jnp
from jax import lax
from jax.experimental import pallas as pl
from jax.experimental.pallas import tpu as pltpu
from jax.experimental.pallas import tpu_sc as plsc

_VOCAB = 100000
_EMBED = 64
_BATCH = 1024
_BLOCK_V = 4096
_NBLK = (_VOCAB + _BLOCK_V - 1) // _BLOCK_V


def _sc_gather(target, in_emb):
    info = plsc.get_sparse_core_info()
    n_workers = info.num_cores * info.num_subcores
    bpw = _BATCH // n_workers
    mesh = plsc.VectorSubcoreMesh(core_axis_name="c", subcore_axis_name="s")

    @functools.partial(
        pl.kernel,
        mesh=mesh,
        out_type=jax.ShapeDtypeStruct((_BATCH, _EMBED), jnp.float32),
        scratch_types=[
            pltpu.VMEM((bpw,), jnp.int32),
            pltpu.VMEM((bpw, _EMBED), jnp.float32),
            pltpu.SemaphoreType.DMA,
        ],
        compiler_params=pltpu.CompilerParams(use_tc_tiling_on_sc=False),
    )
    def gather_kernel(table_hbm, idx_hbm, out_hbm, idx_v, rows_v, sem):
        wid = lax.axis_index("s") * info.num_cores + lax.axis_index("c")
        base = wid * bpw
        pltpu.sync_copy(idx_hbm.at[pl.ds(base, bpw)], idx_v)
        pltpu.async_copy(table_hbm.at[idx_v], rows_v, sem).wait()
        pltpu.sync_copy(rows_v, out_hbm.at[pl.ds(base, bpw)])

    return gather_kernel(in_emb, target)


def _tc_logits_t(embed, out_emb):
    def body(w_ref, emb_ref, out_ref):
        out_ref[...] = lax.dot_general(
            w_ref[...],
            emb_ref[...],
            dimension_numbers=(((1,), (1,)), ((), ())),
            preferred_element_type=jnp.float32,
        )

    return pl.pallas_call(
        body,
        grid=(_NBLK,),
        in_specs=[
            pl.BlockSpec((_BLOCK_V, _EMBED), lambda i: (i, 0)),
            pl.BlockSpec((_BATCH, _EMBED), lambda i: (0, 0)),
        ],
        out_specs=pl.BlockSpec((_BLOCK_V, _BATCH), lambda i: (i, 0)),
        out_shape=jax.ShapeDtypeStruct((_VOCAB, _BATCH), jnp.float32),
        compiler_params=pltpu.CompilerParams(
            dimension_semantics=("arbitrary",),
        ),
    )(out_emb, embed)


def kernel(target, in_emb, out_emb):
    embed = _sc_gather(target.astype(jnp.int32), in_emb)
    return _tc_logits_t(embed, out_emb).T


# speedup vs baseline: 1.2081x; 1.1934x over previous
import functools

import jax
import jax.numpy as jnp
from jax import lax
from jax.experimental import pallas as pl
from jax.experimental.pallas import tpu as pltpu
from jax.experimental.pallas import tpu_sc as plsc

_VOCAB = 100000
_EMBED = 64
_BATCH = 1024
_BLOCK_V = 4096
_NBLK = (_VOCAB + _BLOCK_V - 1) // _BLOCK_V


def _sc_gather(target, in_emb):
    info = plsc.get_sparse_core_info()
    n_workers = info.num_cores * info.num_subcores
    bpw = _BATCH // n_workers
    mesh = plsc.VectorSubcoreMesh(core_axis_name="c", subcore_axis_name="s")

    @functools.partial(
        pl.kernel,
        mesh=mesh,
        out_type=jax.ShapeDtypeStruct((_BATCH, _EMBED), jnp.float32),
        scratch_types=[
            pltpu.VMEM((bpw,), jnp.int32),
            pltpu.VMEM((bpw, _EMBED), jnp.float32),
            pltpu.SemaphoreType.DMA,
        ],
        compiler_params=pltpu.CompilerParams(use_tc_tiling_on_sc=False),
    )
    def gather_kernel(table_hbm, idx_hbm, out_hbm, idx_v, rows_v, sem):
        wid = lax.axis_index("s") * info.num_cores + lax.axis_index("c")
        base = wid * bpw
        pltpu.sync_copy(idx_hbm.at[pl.ds(base, bpw)], idx_v)
        pltpu.async_copy(table_hbm.at[idx_v], rows_v, sem).wait()
        pltpu.sync_copy(rows_v, out_hbm.at[pl.ds(base, bpw)])

    return gather_kernel(in_emb, target)


def _tc_logits_t(embed, out_emb_t):
    def body(w_ref, emb_ref, out_ref):
        out_ref[...] = lax.dot_general(
            w_ref[...],
            emb_ref[...],
            dimension_numbers=(((0,), (1,)), ((), ())),
            preferred_element_type=jnp.float32,
        )

    return pl.pallas_call(
        body,
        grid=(_NBLK,),
        in_specs=[
            pl.BlockSpec((_EMBED, _BLOCK_V), lambda i: (0, i)),
            pl.BlockSpec((_BATCH, _EMBED), lambda i: (0, 0)),
        ],
        out_specs=pl.BlockSpec((_BLOCK_V, _BATCH), lambda i: (i, 0)),
        out_shape=jax.ShapeDtypeStruct((_VOCAB, _BATCH), jnp.float32),
        compiler_params=pltpu.CompilerParams(
            dimension_semantics=("arbitrary",),
        ),
    )(out_emb_t, embed)


def kernel(target, in_emb, out_emb):
    embed = _sc_gather(target.astype(jnp.int32), in_emb)
    return _tc_logits_t(embed, out_emb.T).T
